# baseline (speedup 1.0000x reference)
import numpy as np
import jax
import jax.numpy as jnp
from jax.experimental import pallas as pl
from jax.experimental.pallas import tpu as pltpu

_ANCHOR_SIZES = [128, 256, 512]
_ANCHOR_RATIOS = [0.5, 1, 2]
_STRIDE = 32
_N_TOP = 2000
_N_PAD = 2048
_MAX_OUT = 300
_KEEP_PAD = 384
_NMS_THRESH = 0.7


def _anchor_table():
    flat = []
    for y in range(8):
        for x in range(8):
            for size in _ANCHOR_SIZES:
                for ratio in _ANCHOR_RATIOS:
                    cx, cy = _STRIDE * (x + 0.5), _STRIDE * (y + 0.5)
                    h = size * np.sqrt(1.0 / ratio)
                    w = size * np.sqrt(ratio)
                    flat.append([cx - 0.5 * w, cy - 0.5 * h,
                                 cx + 0.5 * w, cy + 0.5 * h])
    flat = np.array(flat, np.float32)
    A = np.zeros((64, 9, 4), np.float32)
    for a in range(9):
        for pos in range(64):
            box = flat[a * 64 + pos]
            A[pos, a] = (box[0], box[1], box[2] - box[0], box[3] - box[1])
    return A.reshape(64, 36)


def _rpn_kernel(xp_ref, w_ref, cb_ref, cw_ref, clb_ref, bw_ref, bbb_ref,
                anc_ref, scores_ref, pred_ref, acc_ref):
    k = pl.program_id(0)

    @pl.when(k == 0)
    def _init():
        acc_ref[...] = jnp.zeros_like(acc_ref)

    s = jnp.zeros((256, 512), jnp.float32)
    for dy in range(3):
        for dx in range(3):
            xs = xp_ref[:, dy:dy + 8, dx:dx + 8, :].reshape(256, 128)
            wk = w_ref[dy, dx]
            s = s + jax.lax.dot_general(
                xs, wk, (((1,), (1,)), ((), ())),
                preferred_element_type=jnp.float32)
    acc_ref[...] += s

    @pl.when(k == pl.num_programs(0) - 1)
    def _finish():
        feat = acc_ref[...] + cb_ref[...]
        obj = jax.lax.dot_general(
            feat, cw_ref[...], (((1,), (0,)), ((), ())),
            preferred_element_type=jnp.float32) + clb_ref[...]
        dl = jax.lax.dot_general(
            feat, bw_ref[...], (((1,), (0,)), ((), ())),
            preferred_element_type=jnp.float32) + bbb_ref[...]
        scores_ref[...] = obj

        A = anc_ref[...].reshape(256, 9, 4)
        d = dl.reshape(256, 9, 4)
        ex = A[..., 0]
        ey = A[..., 1]
        ew = A[..., 2]
        eh = A[..., 3]
        pcx = ex + d[..., 0] * ew
        pcy = ey + d[..., 1] * eh
        pw = jnp.exp(d[..., 2]) * ew
        ph = jnp.exp(d[..., 3]) * eh
        pred = jnp.stack([pcx - 0.5 * pw, pcy - 0.5 * ph,
                          pcx + 0.5 * pw, pcy + 0.5 * ph], axis=-1)
        pred_ref[...] = pred.reshape(256, 36)


def _nms_kernel(boxes_ref, keep_ref):
    st0 = boxes_ref[...]
    x1 = st0[0:1]
    y1 = st0[1:2]
    x2 = st0[2:3]
    y2 = st0[3:4]
    areas = (x2 - x1) * (y2 - y1)
    lane = jax.lax.broadcasted_iota(jnp.int32, (1, _N_PAD), 1)
    lane_k = jax.lax.broadcasted_iota(jnp.int32, (1, _KEEP_PAD), 1)
    row4 = (jax.lax.broadcasted_iota(jnp.int32, (8, 1), 0) == 4
            ).astype(jnp.float32)

    def body(i, carry):
        st, keepv, cnt = carry
        sel = (lane == i).astype(jnp.float32)
        v = jnp.sum(st * sel, axis=1, keepdims=True)
        xi1 = v[0:1]
        yi1 = v[1:2]
        xi2 = v[2:3]
        yi2 = v[3:4]
        ai = (xi2 - xi1) * (yi2 - yi1)
        valid = v[4:5] < 0.5
        xx1 = jnp.maximum(x1, xi1)
        yy1 = jnp.maximum(y1, yi1)
        xx2 = jnp.minimum(x2, xi2)
        yy2 = jnp.minimum(y2, yi2)
        w = jnp.maximum(xx2 - xx1, 0.0)
        h = jnp.maximum(yy2 - yy1, 0.0)
        inter = w * h
        iou = inter / (areas + ai - inter + 1e-9)
        do_keep = valid & (cnt < _MAX_OUT)
        keepv = jnp.where(do_keep & (lane_k == cnt), i, keepv)
        cnt = cnt + jnp.where(valid, jnp.int32(1), jnp.int32(0))
        newsup = (valid & (iou > _NMS_THRESH) & (lane > i)).astype(jnp.float32)
        st = jnp.maximum(st, newsup * row4)
        return st, keepv, cnt

    keep0 = jnp.zeros((1, _KEEP_PAD), jnp.int32)
    _, keepv, _ = jax.lax.fori_loop(
        0, _N_TOP, body, (st0, keep0, jnp.zeros((1, 1), jnp.int32)))
    keep_ref[...] = keepv


@jax.jit
def kernel(x, conv_w, conv_b, cls_w, cls_b, bbox_w, bbox_b):
    xpad = jnp.pad(x.transpose(0, 2, 3, 1),
                   ((0, 0), (1, 1), (1, 1), (0, 0)))
    w2 = conv_w.transpose(2, 3, 0, 1)
    cw = cls_w.reshape(18, 512).T
    bw = bbox_w.reshape(36, 512).T
    anc = jnp.asarray(np.tile(_anchor_table(), (4, 1)))

    scores2, pred2 = pl.pallas_call(
        _rpn_kernel,
        grid=(16,),
        in_specs=[
            pl.BlockSpec((4, 10, 10, 128), lambda k: (0, 0, 0, k)),
            pl.BlockSpec((3, 3, 512, 128), lambda k: (0, 0, 0, k)),
            pl.BlockSpec((1, 512), lambda k: (0, 0)),
            pl.BlockSpec((512, 18), lambda k: (0, 0)),
            pl.BlockSpec((1, 18), lambda k: (0, 0)),
            pl.BlockSpec((512, 36), lambda k: (0, 0)),
            pl.BlockSpec((1, 36), lambda k: (0, 0)),
            pl.BlockSpec((256, 36), lambda k: (0, 0)),
        ],
        out_specs=[pl.BlockSpec((256, 18), lambda k: (0, 0)),
                   pl.BlockSpec((256, 36), lambda k: (0, 0))],
        out_shape=[jax.ShapeDtypeStruct((256, 18), jnp.float32),
                   jax.ShapeDtypeStruct((256, 36), jnp.float32)],
        scratch_shapes=[pltpu.VMEM((256, 512), jnp.float32)],
    )(xpad, w2, conv_b.reshape(1, 512), cw, cls_b.reshape(1, 18),
      bw, bbox_b.reshape(1, 36), anc)

    scores = scores2.reshape(4, 64, 18).transpose(0, 2, 1).reshape(-1)
    pred = pred2.reshape(4, 64, 36).transpose(0, 2, 1).reshape(4, 36, 8, 8)

    anchors_flat = pred.reshape(-1, 4)
    _, top_idx = jax.lax.top_k(scores, _N_TOP)
    tb = anchors_flat[jnp.clip(top_idx, 0, anchors_flat.shape[0] - 1)]
    boxes_t = jnp.zeros((8, _N_PAD), jnp.float32).at[:4, :_N_TOP].set(tb.T)

    keepv = pl.pallas_call(
        _nms_kernel,
        out_shape=jax.ShapeDtypeStruct((1, _KEEP_PAD), jnp.int32),
    )(boxes_t)
    keep = keepv[0, :_MAX_OUT]
    final = pred[jnp.clip(keep, 0, 3)]
    return final


# speedup vs baseline: 1.0659x; 1.0659x over previous
import numpy as np
import jax
import jax.numpy as jnp
from jax.experimental import pallas as pl
from jax.experimental.pallas import tpu as pltpu

_ANCHOR_SIZES = [128, 256, 512]
_ANCHOR_RATIOS = [0.5, 1, 2]
_STRIDE = 32
_N_TOP = 2000
_N_PAD = 2048
_MAX_OUT = 300
_KEEP_PAD = 384
_NMS_THRESH = 0.7


def _anchor_table():
    flat = []
    for y in range(8):
        for x in range(8):
            for size in _ANCHOR_SIZES:
                for ratio in _ANCHOR_RATIOS:
                    cx, cy = _STRIDE * (x + 0.5), _STRIDE * (y + 0.5)
                    h = size * np.sqrt(1.0 / ratio)
                    w = size * np.sqrt(ratio)
                    flat.append([cx - 0.5 * w, cy - 0.5 * h,
                                 cx + 0.5 * w, cy + 0.5 * h])
    flat = np.array(flat, np.float32)
    A = np.zeros((64, 9, 4), np.float32)
    for a in range(9):
        for pos in range(64):
            box = flat[a * 64 + pos]
            A[pos, a] = (box[0], box[1], box[2] - box[0], box[3] - box[1])
    return A.reshape(64, 36)


def _rpn_kernel(xp_ref, w_ref, cb_ref, cw_ref, clb_ref, bw_ref, bbb_ref,
                anc_ref, scores_ref, pred_ref, acc_ref):
    k = pl.program_id(0)

    @pl.when(k == 0)
    def _init():
        acc_ref[...] = jnp.zeros_like(acc_ref)

    s = jnp.zeros((256, 512), jnp.float32)
    for dy in range(3):
        for dx in range(3):
            xs = xp_ref[:, dy:dy + 8, dx:dx + 8, :].reshape(256, 128)
            wk = w_ref[dy, dx]
            s = s + jax.lax.dot_general(
                xs, wk, (((1,), (1,)), ((), ())),
                preferred_element_type=jnp.float32)
    acc_ref[...] += s

    @pl.when(k == pl.num_programs(0) - 1)
    def _finish():
        feat = acc_ref[...] + cb_ref[...]
        obj = jax.lax.dot_general(
            feat, cw_ref[...], (((1,), (0,)), ((), ())),
            preferred_element_type=jnp.float32) + clb_ref[...]
        dl = jax.lax.dot_general(
            feat, bw_ref[...], (((1,), (0,)), ((), ())),
            preferred_element_type=jnp.float32) + bbb_ref[...]
        scores_ref[...] = obj

        A = anc_ref[...].reshape(256, 9, 4)
        d = dl.reshape(256, 9, 4)
        ex = A[..., 0]
        ey = A[..., 1]
        ew = A[..., 2]
        eh = A[..., 3]
        pcx = ex + d[..., 0] * ew
        pcy = ey + d[..., 1] * eh
        pw = jnp.exp(d[..., 2]) * ew
        ph = jnp.exp(d[..., 3]) * eh
        pred = jnp.stack([pcx - 0.5 * pw, pcy - 0.5 * ph,
                          pcx + 0.5 * pw, pcy + 0.5 * ph], axis=-1)
        pred_ref[...] = pred.reshape(256, 36)


def _nms_kernel(boxes_ref, keep_ref):
    b = boxes_ref[...]
    x1 = b[0:1, :]
    y1 = b[1:2, :]
    x2 = b[2:3, :]
    y2 = b[3:4, :]
    areas = (x2 - x1) * (y2 - y1)
    lane = jax.lax.broadcasted_iota(jnp.int32, (1, _N_PAD), 1)
    lane_k = jax.lax.broadcasted_iota(jnp.int32, (1, _KEEP_PAD), 1)

    def body(i, carry):
        supp, keepv, cnt = carry
        sel = (lane == i).astype(jnp.float32)
        xi1 = jnp.sum(x1 * sel)
        yi1 = jnp.sum(y1 * sel)
        xi2 = jnp.sum(x2 * sel)
        yi2 = jnp.sum(y2 * sel)
        ai = (xi2 - xi1) * (yi2 - yi1)
        si = jnp.sum(supp * sel)
        valid = si < 0.5
        xx1 = jnp.maximum(x1, xi1)
        yy1 = jnp.maximum(y1, yi1)
        xx2 = jnp.minimum(x2, xi2)
        yy2 = jnp.minimum(y2, yi2)
        w = jnp.maximum(xx2 - xx1, 0.0)
        h = jnp.maximum(yy2 - yy1, 0.0)
        inter = w * h
        iou = inter / (areas + ai - inter + 1e-9)
        do_keep = valid & (cnt < _MAX_OUT)
        keepv = jnp.where(do_keep & (lane_k == cnt), i, keepv)
        cnt = cnt + jnp.where(valid, jnp.int32(1), jnp.int32(0))
        newsup = (valid & (iou > _NMS_THRESH) & (lane > i)).astype(jnp.float32)
        supp = jnp.maximum(supp, newsup)
        return supp, keepv, cnt

    supp0 = jnp.zeros((1, _N_PAD), jnp.float32)
    keep0 = jnp.zeros((1, _KEEP_PAD), jnp.int32)
    _, keepv, _ = jax.lax.fori_loop(0, _N_TOP, body,
                                    (supp0, keep0, jnp.int32(0)))
    keep_ref[...] = keepv


@jax.jit
def kernel(x, conv_w, conv_b, cls_w, cls_b, bbox_w, bbox_b):
    xpad = jnp.pad(x.transpose(0, 2, 3, 1),
                   ((0, 0), (1, 1), (1, 1), (0, 0)))
    w2 = conv_w.transpose(2, 3, 0, 1)
    cw = cls_w.reshape(18, 512).T
    bw = bbox_w.reshape(36, 512).T
    anc = jnp.asarray(np.tile(_anchor_table(), (4, 1)))

    scores2, pred2 = pl.pallas_call(
        _rpn_kernel,
        grid=(16,),
        in_specs=[
            pl.BlockSpec((4, 10, 10, 128), lambda k: (0, 0, 0, k)),
            pl.BlockSpec((3, 3, 512, 128), lambda k: (0, 0, 0, k)),
            pl.BlockSpec((1, 512), lambda k: (0, 0)),
            pl.BlockSpec((512, 18), lambda k: (0, 0)),
            pl.BlockSpec((1, 18), lambda k: (0, 0)),
            pl.BlockSpec((512, 36), lambda k: (0, 0)),
            pl.BlockSpec((1, 36), lambda k: (0, 0)),
            pl.BlockSpec((256, 36), lambda k: (0, 0)),
        ],
        out_specs=[pl.BlockSpec((256, 18), lambda k: (0, 0)),
                   pl.BlockSpec((256, 36), lambda k: (0, 0))],
        out_shape=[jax.ShapeDtypeStruct((256, 18), jnp.float32),
                   jax.ShapeDtypeStruct((256, 36), jnp.float32)],
        scratch_shapes=[pltpu.VMEM((256, 512), jnp.float32)],
    )(xpad, w2, conv_b.reshape(1, 512), cw, cls_b.reshape(1, 18),
      bw, bbox_b.reshape(1, 36), anc)

    scores = scores2.reshape(4, 64, 18).transpose(0, 2, 1).reshape(-1)
    pred = pred2.reshape(4, 64, 36).transpose(0, 2, 1).reshape(4, 36, 8, 8)

    anchors_flat = pred.reshape(-1, 4)
    _, top_idx = jax.lax.top_k(scores, _N_TOP)
    tb = anchors_flat[jnp.clip(top_idx, 0, anchors_flat.shape[0] - 1)]
    boxes_t = jnp.zeros((4, _N_PAD), jnp.float32).at[:, :_N_TOP].set(tb.T)

    keepv = pl.pallas_call(
        _nms_kernel,
        out_shape=jax.ShapeDtypeStruct((1, _KEEP_PAD), jnp.int32),
    )(boxes_t)
    keep = keepv[0, :_MAX_OUT]
    final = pred[jnp.clip(keep, 0, 3)]
    return final
